# baseline (speedup 1.0000x reference)
import functools

import jax
import jax.numpy as jnp
from jax import lax
from jax.experimental import pallas as pl
from jax.experimental.pallas import tpu as pltpu
from jax.experimental.pallas import tpu_sc as plsc

N = 4096
L = 16
NC = 2
NS = 16
NW = NC * NS
S_TC = 3584
N_SC = N - S_TC
RPW = N_SC // NW
NCHUNK = N // L
BIG = 3.0e38


def _prep_body(pose_ref, kb_ref, pr_ref, coords_ref, gt3_ref, diag_ref):
    coords_ref[0, :] = pr_ref[0, :]
    coords_ref[1, :] = pr_ref[1, :]
    coords_ref[2, :] = pr_ref[2, :]
    for blk in range(32):
        b = blk // 8
        s = blk * 128
        kx = kb_ref[0, pl.ds(s, 128)]
        ky = kb_ref[1, pl.ds(s, 128)]
        kz = kb_ref[2, pl.ds(s, 128)]
        g = [None] * 3
        for d in range(3):
            g[d] = (pose_ref[b, d * 4 + 0] * kx
                    + pose_ref[b, d * 4 + 1] * ky
                    + pose_ref[b, d * 4 + 2] * kz
                    + pose_ref[b, d * 4 + 3])
            coords_ref[3 + d, pl.ds(s, 128)] = g[d]
            gt3_ref[d, blk, :] = g[d]
        dx = pr_ref[0, pl.ds(s, 128)] - g[0]
        dy = pr_ref[1, pl.ds(s, 128)] - g[1]
        dz = pr_ref[2, pl.ds(s, 128)] - g[2]
        diag_ref[0, blk, :] = dx * dx + dy * dy + dz * dz
        diag_ref[1, blk, :] = jnp.abs(dx) + jnp.abs(dy) + jnp.abs(dz)


_prep_call = pl.pallas_call(
    _prep_body,
    out_shape=[
        jax.ShapeDtypeStruct((6, N), jnp.float32),
        jax.ShapeDtypeStruct((3, 32, 128), jnp.float32),
        jax.ShapeDtypeStruct((2, 32, 128), jnp.float32),
    ],
    in_specs=[
        pl.BlockSpec(memory_space=pltpu.SMEM),
        pl.BlockSpec(memory_space=pltpu.VMEM),
        pl.BlockSpec(memory_space=pltpu.VMEM),
    ],
)


def _sc_body(coords, minlt_hbm, mingt_hbm, gx, gy, gz, px, py, pz, lt_o, gt_o):
    wid = lax.axis_index("s") * NC + lax.axis_index("c")
    out_base = wid * RPW
    base = S_TC + out_base
    pltpu.sync_copy(coords.at[3], gx)
    pltpu.sync_copy(coords.at[4], gy)
    pltpu.sync_copy(coords.at[5], gz)
    pltpu.sync_copy(coords.at[0, pl.ds(base, RPW)], px)
    pltpu.sync_copy(coords.at[1, pl.ds(base, RPW)], py)
    pltpu.sync_copy(coords.at[2, pl.ds(base, RPW)], pz)

    lane = lax.iota(jnp.int32, L)
    big = jnp.full((L,), BIG, jnp.float32)
    zeros = jnp.zeros((L,), jnp.float32)

    def group_fn(g, carry):
        gbase = g * L
        pxc = px[pl.ds(gbase, L)]
        pyc = py[pl.ds(gbase, L)]
        pzc = pz[pl.ds(gbase, L)]
        cb = (base + gbase) // L

        def row_fn(rr, rcarry):
            res_lt, res_gt = rcarry
            sel = lane == rr
            pxs = jnp.full((L,), jnp.sum(jnp.where(sel, pxc, 0.0)))
            pys = jnp.full((L,), jnp.sum(jnp.where(sel, pyc, 0.0)))
            pzs = jnp.full((L,), jnp.sum(jnp.where(sel, pzc, 0.0)))

            def dsq_at(c):
                gxv = gx[pl.ds(c * L, L)]
                gyv = gy[pl.ds(c * L, L)]
                gzv = gz[pl.ds(c * L, L)]
                dx = pxs - gxv
                dy = pys - gyv
                dz = pzs - gzv
                return dx * dx + dy * dy + dz * dz

            def mn(c, acc):
                return jnp.minimum(acc, dsq_at(c))

            acc_lt = plsc.parallel_loop(0, cb, unroll=8, carry=big)(mn)
            acc_gt = plsc.parallel_loop(cb + 1, NCHUNK, unroll=8, carry=big)(mn)
            db = dsq_at(cb)
            acc_lt = jnp.minimum(acc_lt, jnp.where(lane < rr, db, BIG))
            acc_gt = jnp.minimum(acc_gt, jnp.where(lane > rr, db, BIG))
            mlt = jnp.min(acc_lt)
            mgt = jnp.min(acc_gt)
            res_lt = jnp.where(sel, mlt, res_lt)
            res_gt = jnp.where(sel, mgt, res_gt)
            return res_lt, res_gt

        res_lt, res_gt = lax.fori_loop(0, L, row_fn, (zeros, zeros))
        lt_o[pl.ds(gbase, L)] = res_lt
        gt_o[pl.ds(gbase, L)] = res_gt
        return carry

    lax.fori_loop(0, RPW // L, group_fn, 0)
    orow = out_base // 128
    ocol = out_base % 128
    pltpu.sync_copy(lt_o, minlt_hbm.at[orow, pl.ds(ocol, RPW)])
    pltpu.sync_copy(gt_o, mingt_hbm.at[orow, pl.ds(ocol, RPW)])


@functools.cache
def _get_sc_call():
    mesh = plsc.VectorSubcoreMesh(core_axis_name="c", subcore_axis_name="s")
    return functools.partial(
        pl.kernel,
        out_type=[
            jax.ShapeDtypeStruct((N_SC // 128, 128), jnp.float32),
            jax.ShapeDtypeStruct((N_SC // 128, 128), jnp.float32),
        ],
        mesh=mesh,
        compiler_params=pltpu.CompilerParams(needs_layout_passes=False),
        scratch_types=[
            pltpu.VMEM((N,), jnp.float32),
            pltpu.VMEM((N,), jnp.float32),
            pltpu.VMEM((N,), jnp.float32),
            pltpu.VMEM((RPW,), jnp.float32),
            pltpu.VMEM((RPW,), jnp.float32),
            pltpu.VMEM((RPW,), jnp.float32),
            pltpu.VMEM((RPW,), jnp.float32),
            pltpu.VMEM((RPW,), jnp.float32),
        ],
    )(_sc_body)


def _tcdist_body(pred_ref, gt_ref, lt_ref, gt_out_ref):
    t = pl.program_id(0)
    sub = lax.broadcasted_iota(jnp.int32, (8, 128), 0)
    lanei = lax.broadcasted_iota(jnp.int32, (8, 128), 1)
    big = jnp.full((8, 128), BIG, jnp.float32)

    def pred_block(rb):
        out = []
        for d in range(3):
            rows = [pred_ref[d, t * 128 + rb * 8 + s] for s in range(8)]
            out.append(jnp.concatenate(
                [jnp.full((1, 128), v, jnp.float32) for v in rows], axis=0))
        return tuple(out)

    def gt_chunk(c):
        gxv = jnp.broadcast_to(gt_ref[0, c, :].reshape(1, 128), (8, 128))
        gyv = jnp.broadcast_to(gt_ref[1, c, :].reshape(1, 128), (8, 128))
        gzv = jnp.broadcast_to(gt_ref[2, c, :].reshape(1, 128), (8, 128))
        return gxv, gyv, gzv

    def dsq(p, g):
        dx = p[0] - g[0]
        dy = p[1] - g[1]
        dz = p[2] - g[2]
        return dx * dx + dy * dy + dz * dz

    GK = 4
    zero = jnp.float32(0.0)
    bigs = jnp.float32(BIG)
    for rbg in range(16 // GK):
        rbs = [rbg * GK + k for k in range(GK)]
        preds = [pred_block(rb) for rb in rbs]
        accs_lt = [big] * GK
        accs_gt = [big] * GK
        for c in range(N // 128):
            g = gt_chunk(c)
            pen_lt = jnp.where(c < t, zero, bigs)
            pen_gt = jnp.where(c > t, zero, bigs)
            for k in range(GK):
                d = dsq(preds[k], g)
                accs_lt[k] = jnp.minimum(accs_lt[k], d + pen_lt)
                accs_gt[k] = jnp.minimum(accs_gt[k], d + pen_gt)
        gb = gt_chunk(t)
        for k, rb in enumerate(rbs):
            db = dsq(preds[k], gb)
            acc_lt = jnp.minimum(accs_lt[k],
                                 jnp.where(lanei < rb * 8 + sub, db, BIG))
            acc_gt = jnp.minimum(accs_gt[k],
                                 jnp.where(lanei > rb * 8 + sub, db, BIG))
            lt_ref[t, pl.ds(rb * 8, 8)] = jnp.min(acc_lt, axis=1)
            gt_out_ref[t, pl.ds(rb * 8, 8)] = jnp.min(acc_gt, axis=1)


_tcdist_call = pl.pallas_call(
    _tcdist_body,
    grid=(S_TC // 128,),
    in_specs=[
        pl.BlockSpec(memory_space=pltpu.SMEM),
        pl.BlockSpec(memory_space=pltpu.VMEM),
    ],
    out_specs=[
        pl.BlockSpec((S_TC // 128, 128), lambda t: (0, 0)),
        pl.BlockSpec((S_TC // 128, 128), lambda t: (0, 0)),
    ],
    out_shape=[
        jax.ShapeDtypeStruct((S_TC // 128, 128), jnp.float32),
        jax.ShapeDtypeStruct((S_TC // 128, 128), jnp.float32),
    ],
)


def _post_body(mlt_tc_ref, mgt_tc_ref, mlt_sc_ref, mgt_sc_ref,
               diag_ref, w_ref, lg_ref, out_ref):
    ntc = S_TC // 128
    s_ii_tc = jnp.sqrt(diag_ref[0, :ntc])
    m_tc = jnp.logical_and(jnp.sqrt(mlt_tc_ref[...]) > s_ii_tc,
                           jnp.sqrt(mgt_tc_ref[...]) >= s_ii_tc)
    s_ii_sc = jnp.sqrt(diag_ref[0, ntc:])
    m_sc = jnp.logical_and(jnp.sqrt(mlt_sc_ref[...]) > s_ii_sc,
                           jnp.sqrt(mgt_sc_ref[...]) >= s_ii_sc)
    m1 = m_tc.astype(jnp.float32)
    m2 = m_sc.astype(jnp.float32)
    x = lg_ref[...]
    bce0 = jnp.maximum(x, 0.0) + jnp.log(1.0 + jnp.exp(-jnp.abs(x)))
    bce1 = bce0 - x
    cnt1 = jnp.sum(m1) + jnp.sum(m2)
    cnt0 = jnp.float32(N) - cnt1
    s0 = (jnp.sum(bce0[:ntc] * (1.0 - m1))
          + jnp.sum(bce0[ntc:] * (1.0 - m2)))
    s1 = jnp.sum(bce1[:ntc] * m1) + jnp.sum(bce1[ntc:] * m2)
    mean0 = s0 / jnp.maximum(cnt0, 1.0)
    mean1 = s1 / jnp.maximum(cnt1, 1.0)
    inlier = (jnp.where(cnt0 > 0.0, mean0, 0.0)
              + jnp.where(cnt1 > 0.0, mean1, 0.0)) * 0.5
    w = w_ref[...]
    err = diag_ref[1]
    mean_err = jnp.sum(w * err) / jnp.maximum(jnp.sum(w), 1e-6)
    out_ref[0, 0] = mean_err + inlier


_post_call = pl.pallas_call(
    _post_body,
    out_shape=jax.ShapeDtypeStruct((1, 1), jnp.float32),
    out_specs=pl.BlockSpec(memory_space=pltpu.SMEM),
)


def kernel(kp_before, kp_warped_pred, pose_gt, overlap_weights, inlier_logits):
    kb = jnp.transpose(kp_before, (2, 0, 1)).reshape(3, N)
    pr = jnp.transpose(kp_warped_pred, (2, 0, 1)).reshape(3, N)
    pose = pose_gt.reshape(4, 12)
    coords, gt3, diag = _prep_call(pose, kb, pr)
    minlt_sc, mingt_sc = _get_sc_call()(coords)
    minlt_tc, mingt_tc = _tcdist_call(pr, gt3)
    out = _post_call(
        minlt_tc, mingt_tc, minlt_sc, mingt_sc, diag,
        overlap_weights.reshape(32, 128),
        inlier_logits.reshape(32, 128),
    )
    return out[0, 0]


# speedup vs baseline: 1.1160x; 1.1160x over previous
import functools

import jax
import jax.numpy as jnp
from jax import lax
from jax.experimental import pallas as pl
from jax.experimental.pallas import tpu as pltpu
from jax.experimental.pallas import tpu_sc as plsc

N = 4096
L = 16
NC = 2
NS = 16
NW = NC * NS
S_TC = 3072
TPG = 2
N_SC = N - S_TC
RPW = N_SC // NW
NCHUNK = N // L
BIG = 3.0e38


def _prep_body(pose_ref, kb_ref, pr_ref, coords_ref, gt3_ref, diag_ref):
    coords_ref[0, :] = pr_ref[0, :]
    coords_ref[1, :] = pr_ref[1, :]
    coords_ref[2, :] = pr_ref[2, :]
    for blk in range(32):
        b = blk // 8
        s = blk * 128
        kx = kb_ref[0, pl.ds(s, 128)]
        ky = kb_ref[1, pl.ds(s, 128)]
        kz = kb_ref[2, pl.ds(s, 128)]
        g = [None] * 3
        for d in range(3):
            g[d] = (pose_ref[b, d * 4 + 0] * kx
                    + pose_ref[b, d * 4 + 1] * ky
                    + pose_ref[b, d * 4 + 2] * kz
                    + pose_ref[b, d * 4 + 3])
            coords_ref[3 + d, pl.ds(s, 128)] = g[d]
            gt3_ref[d, blk, :] = g[d]
        dx = pr_ref[0, pl.ds(s, 128)] - g[0]
        dy = pr_ref[1, pl.ds(s, 128)] - g[1]
        dz = pr_ref[2, pl.ds(s, 128)] - g[2]
        diag_ref[0, blk, :] = dx * dx + dy * dy + dz * dz
        diag_ref[1, blk, :] = jnp.abs(dx) + jnp.abs(dy) + jnp.abs(dz)


_prep_call = pl.pallas_call(
    _prep_body,
    out_shape=[
        jax.ShapeDtypeStruct((6, N), jnp.float32),
        jax.ShapeDtypeStruct((3, 32, 128), jnp.float32),
        jax.ShapeDtypeStruct((2, 32, 128), jnp.float32),
    ],
    in_specs=[
        pl.BlockSpec(memory_space=pltpu.SMEM),
        pl.BlockSpec(memory_space=pltpu.VMEM),
        pl.BlockSpec(memory_space=pltpu.VMEM),
    ],
)


def _sc_body(coords, minlt_hbm, mingt_hbm, gx, gy, gz, px, py, pz, lt_o, gt_o):
    wid = lax.axis_index("s") * NC + lax.axis_index("c")
    out_base = wid * RPW
    base = S_TC + out_base
    pltpu.sync_copy(coords.at[3], gx)
    pltpu.sync_copy(coords.at[4], gy)
    pltpu.sync_copy(coords.at[5], gz)
    pltpu.sync_copy(coords.at[0, pl.ds(base, RPW)], px)
    pltpu.sync_copy(coords.at[1, pl.ds(base, RPW)], py)
    pltpu.sync_copy(coords.at[2, pl.ds(base, RPW)], pz)

    lane = lax.iota(jnp.int32, L)
    big = jnp.full((L,), BIG, jnp.float32)
    zeros = jnp.zeros((L,), jnp.float32)

    def group_fn(g, carry):
        gbase = g * L
        pxc = px[pl.ds(gbase, L)]
        pyc = py[pl.ds(gbase, L)]
        pzc = pz[pl.ds(gbase, L)]
        cb = (base + gbase) // L

        def row_fn(rr, rcarry):
            res_lt, res_gt = rcarry
            sel = lane == rr
            pxs = jnp.full((L,), jnp.sum(jnp.where(sel, pxc, 0.0)))
            pys = jnp.full((L,), jnp.sum(jnp.where(sel, pyc, 0.0)))
            pzs = jnp.full((L,), jnp.sum(jnp.where(sel, pzc, 0.0)))

            def dsq_at(c):
                gxv = gx[pl.ds(c * L, L)]
                gyv = gy[pl.ds(c * L, L)]
                gzv = gz[pl.ds(c * L, L)]
                dx = pxs - gxv
                dy = pys - gyv
                dz = pzs - gzv
                return dx * dx + dy * dy + dz * dz

            def mn(c, acc):
                return jnp.minimum(acc, dsq_at(c))

            acc_lt = plsc.parallel_loop(0, cb, unroll=8, carry=big)(mn)
            acc_gt = plsc.parallel_loop(cb + 1, NCHUNK, unroll=8, carry=big)(mn)
            db = dsq_at(cb)
            acc_lt = jnp.minimum(acc_lt, jnp.where(lane < rr, db, BIG))
            acc_gt = jnp.minimum(acc_gt, jnp.where(lane > rr, db, BIG))
            mlt = jnp.min(acc_lt)
            mgt = jnp.min(acc_gt)
            res_lt = jnp.where(sel, mlt, res_lt)
            res_gt = jnp.where(sel, mgt, res_gt)
            return res_lt, res_gt

        res_lt, res_gt = lax.fori_loop(0, L, row_fn, (zeros, zeros))
        lt_o[pl.ds(gbase, L)] = res_lt
        gt_o[pl.ds(gbase, L)] = res_gt
        return carry

    lax.fori_loop(0, RPW // L, group_fn, 0)
    orow = out_base // 128
    ocol = out_base % 128
    pltpu.sync_copy(lt_o, minlt_hbm.at[orow, pl.ds(ocol, RPW)])
    pltpu.sync_copy(gt_o, mingt_hbm.at[orow, pl.ds(ocol, RPW)])


@functools.cache
def _get_sc_call():
    mesh = plsc.VectorSubcoreMesh(core_axis_name="c", subcore_axis_name="s")
    return functools.partial(
        pl.kernel,
        out_type=[
            jax.ShapeDtypeStruct((N_SC // 128, 128), jnp.float32),
            jax.ShapeDtypeStruct((N_SC // 128, 128), jnp.float32),
        ],
        mesh=mesh,
        compiler_params=pltpu.CompilerParams(needs_layout_passes=False),
        scratch_types=[
            pltpu.VMEM((N,), jnp.float32),
            pltpu.VMEM((N,), jnp.float32),
            pltpu.VMEM((N,), jnp.float32),
            pltpu.VMEM((RPW,), jnp.float32),
            pltpu.VMEM((RPW,), jnp.float32),
            pltpu.VMEM((RPW,), jnp.float32),
            pltpu.VMEM((RPW,), jnp.float32),
            pltpu.VMEM((RPW,), jnp.float32),
        ],
    )(_sc_body)


def _tcdist_body(pred_ref, gt_ref, lt_ref, gt_out_ref):
    tg = pl.program_id(0)
    sub = lax.broadcasted_iota(jnp.int32, (8, 128), 0)
    lanei = lax.broadcasted_iota(jnp.int32, (8, 128), 1)
    big = jnp.full((8, 128), BIG, jnp.float32)

    for tile_sub in range(TPG):
        t = tg * TPG + tile_sub
        _tcdist_tile(pred_ref, gt_ref, lt_ref, gt_out_ref, t, sub, lanei, big)


def _tcdist_tile(pred_ref, gt_ref, lt_ref, gt_out_ref, t, sub, lanei, big):
    def pred_block(rb):
        out = []
        for d in range(3):
            rows = [pred_ref[d, t * 128 + rb * 8 + s] for s in range(8)]
            out.append(jnp.concatenate(
                [jnp.full((1, 128), v, jnp.float32) for v in rows], axis=0))
        return tuple(out)

    def gt_chunk(c):
        gxv = jnp.broadcast_to(gt_ref[0, c, :].reshape(1, 128), (8, 128))
        gyv = jnp.broadcast_to(gt_ref[1, c, :].reshape(1, 128), (8, 128))
        gzv = jnp.broadcast_to(gt_ref[2, c, :].reshape(1, 128), (8, 128))
        return gxv, gyv, gzv

    def dsq(p, g):
        dx = p[0] - g[0]
        dy = p[1] - g[1]
        dz = p[2] - g[2]
        return dx * dx + dy * dy + dz * dz

    GK = 4
    zero = jnp.float32(0.0)
    bigs = jnp.float32(BIG)
    for rbg in range(16 // GK):
        rbs = [rbg * GK + k for k in range(GK)]
        preds = [pred_block(rb) for rb in rbs]
        accs_lt = [big] * GK
        accs_gt = [big] * GK
        for c in range(N // 128):
            g = gt_chunk(c)
            pen_lt = jnp.where(c < t, zero, bigs)
            pen_gt = jnp.where(c > t, zero, bigs)
            for k in range(GK):
                d = dsq(preds[k], g)
                accs_lt[k] = jnp.minimum(accs_lt[k], d + pen_lt)
                accs_gt[k] = jnp.minimum(accs_gt[k], d + pen_gt)
        gb = gt_chunk(t)
        for k, rb in enumerate(rbs):
            db = dsq(preds[k], gb)
            acc_lt = jnp.minimum(accs_lt[k],
                                 jnp.where(lanei < rb * 8 + sub, db, BIG))
            acc_gt = jnp.minimum(accs_gt[k],
                                 jnp.where(lanei > rb * 8 + sub, db, BIG))
            lt_ref[t, pl.ds(rb * 8, 8)] = jnp.min(acc_lt, axis=1)
            gt_out_ref[t, pl.ds(rb * 8, 8)] = jnp.min(acc_gt, axis=1)


_tcdist_call = pl.pallas_call(
    _tcdist_body,
    grid=(S_TC // (128 * TPG),),
    in_specs=[
        pl.BlockSpec(memory_space=pltpu.SMEM),
        pl.BlockSpec(memory_space=pltpu.VMEM),
    ],
    out_specs=[
        pl.BlockSpec((S_TC // 128, 128), lambda t: (0, 0)),
        pl.BlockSpec((S_TC // 128, 128), lambda t: (0, 0)),
    ],
    out_shape=[
        jax.ShapeDtypeStruct((S_TC // 128, 128), jnp.float32),
        jax.ShapeDtypeStruct((S_TC // 128, 128), jnp.float32),
    ],
)


def _post_body(mlt_tc_ref, mgt_tc_ref, mlt_sc_ref, mgt_sc_ref,
               diag_ref, w_ref, lg_ref, out_ref):
    ntc = S_TC // 128
    s_ii_tc = jnp.sqrt(diag_ref[0, :ntc])
    m_tc = jnp.logical_and(jnp.sqrt(mlt_tc_ref[...]) > s_ii_tc,
                           jnp.sqrt(mgt_tc_ref[...]) >= s_ii_tc)
    s_ii_sc = jnp.sqrt(diag_ref[0, ntc:])
    m_sc = jnp.logical_and(jnp.sqrt(mlt_sc_ref[...]) > s_ii_sc,
                           jnp.sqrt(mgt_sc_ref[...]) >= s_ii_sc)
    m1 = m_tc.astype(jnp.float32)
    m2 = m_sc.astype(jnp.float32)
    x = lg_ref[...]
    bce0 = jnp.maximum(x, 0.0) + jnp.log(1.0 + jnp.exp(-jnp.abs(x)))
    bce1 = bce0 - x
    cnt1 = jnp.sum(m1) + jnp.sum(m2)
    cnt0 = jnp.float32(N) - cnt1
    s0 = (jnp.sum(bce0[:ntc] * (1.0 - m1))
          + jnp.sum(bce0[ntc:] * (1.0 - m2)))
    s1 = jnp.sum(bce1[:ntc] * m1) + jnp.sum(bce1[ntc:] * m2)
    mean0 = s0 / jnp.maximum(cnt0, 1.0)
    mean1 = s1 / jnp.maximum(cnt1, 1.0)
    inlier = (jnp.where(cnt0 > 0.0, mean0, 0.0)
              + jnp.where(cnt1 > 0.0, mean1, 0.0)) * 0.5
    w = w_ref[...]
    err = diag_ref[1]
    mean_err = jnp.sum(w * err) / jnp.maximum(jnp.sum(w), 1e-6)
    out_ref[0, 0] = mean_err + inlier


_post_call = pl.pallas_call(
    _post_body,
    out_shape=jax.ShapeDtypeStruct((1, 1), jnp.float32),
    out_specs=pl.BlockSpec(memory_space=pltpu.SMEM),
)


def kernel(kp_before, kp_warped_pred, pose_gt, overlap_weights, inlier_logits):
    kb = jnp.transpose(kp_before, (2, 0, 1)).reshape(3, N)
    pr = jnp.transpose(kp_warped_pred, (2, 0, 1)).reshape(3, N)
    pose = pose_gt.reshape(4, 12)
    coords, gt3, diag = _prep_call(pose, kb, pr)
    minlt_sc, mingt_sc = _get_sc_call()(coords)
    minlt_tc, mingt_tc = _tcdist_call(pr, gt3)
    out = _post_call(
        minlt_tc, mingt_tc, minlt_sc, mingt_sc, diag,
        overlap_weights.reshape(32, 128),
        inlier_logits.reshape(32, 128),
    )
    return out[0, 0]


# speedup vs baseline: 1.1300x; 1.0126x over previous
import functools

import jax
import jax.numpy as jnp
from jax import lax
from jax.experimental import pallas as pl
from jax.experimental.pallas import tpu as pltpu
from jax.experimental.pallas import tpu_sc as plsc

N = 4096
L = 16
NC = 2
NS = 16
NW = NC * NS
S_TC = 3072
TPG = 4
N_SC = N - S_TC
RPW = N_SC // NW
NCHUNK = N // L
BIG = 3.0e38


def _prep_body(pose_ref, kb_ref, pr_ref, coords_ref, gt3_ref, diag_ref):
    coords_ref[0, :] = pr_ref[0, :]
    coords_ref[1, :] = pr_ref[1, :]
    coords_ref[2, :] = pr_ref[2, :]
    for blk in range(32):
        b = blk // 8
        s = blk * 128
        kx = kb_ref[0, pl.ds(s, 128)]
        ky = kb_ref[1, pl.ds(s, 128)]
        kz = kb_ref[2, pl.ds(s, 128)]
        g = [None] * 3
        for d in range(3):
            g[d] = (pose_ref[b, d * 4 + 0] * kx
                    + pose_ref[b, d * 4 + 1] * ky
                    + pose_ref[b, d * 4 + 2] * kz
                    + pose_ref[b, d * 4 + 3])
            coords_ref[3 + d, pl.ds(s, 128)] = g[d]
            gt3_ref[d, blk, :] = g[d]
        dx = pr_ref[0, pl.ds(s, 128)] - g[0]
        dy = pr_ref[1, pl.ds(s, 128)] - g[1]
        dz = pr_ref[2, pl.ds(s, 128)] - g[2]
        diag_ref[0, blk, :] = dx * dx + dy * dy + dz * dz
        diag_ref[1, blk, :] = jnp.abs(dx) + jnp.abs(dy) + jnp.abs(dz)


_prep_call = pl.pallas_call(
    _prep_body,
    out_shape=[
        jax.ShapeDtypeStruct((6, N), jnp.float32),
        jax.ShapeDtypeStruct((3, 32, 128), jnp.float32),
        jax.ShapeDtypeStruct((2, 32, 128), jnp.float32),
    ],
    in_specs=[
        pl.BlockSpec(memory_space=pltpu.SMEM),
        pl.BlockSpec(memory_space=pltpu.VMEM),
        pl.BlockSpec(memory_space=pltpu.VMEM),
    ],
)


def _sc_body(coords, minlt_hbm, mingt_hbm, gx, gy, gz, px, py, pz, lt_o, gt_o):
    wid = lax.axis_index("s") * NC + lax.axis_index("c")
    out_base = wid * RPW
    base = S_TC + out_base
    pltpu.sync_copy(coords.at[3], gx)
    pltpu.sync_copy(coords.at[4], gy)
    pltpu.sync_copy(coords.at[5], gz)
    pltpu.sync_copy(coords.at[0, pl.ds(base, RPW)], px)
    pltpu.sync_copy(coords.at[1, pl.ds(base, RPW)], py)
    pltpu.sync_copy(coords.at[2, pl.ds(base, RPW)], pz)

    lane = lax.iota(jnp.int32, L)
    big = jnp.full((L,), BIG, jnp.float32)
    zeros = jnp.zeros((L,), jnp.float32)

    def group_fn(g, carry):
        gbase = g * L
        pxc = px[pl.ds(gbase, L)]
        pyc = py[pl.ds(gbase, L)]
        pzc = pz[pl.ds(gbase, L)]
        cb = (base + gbase) // L

        def row_fn(rr, rcarry):
            res_lt, res_gt = rcarry
            sel = lane == rr
            pxs = jnp.full((L,), jnp.sum(jnp.where(sel, pxc, 0.0)))
            pys = jnp.full((L,), jnp.sum(jnp.where(sel, pyc, 0.0)))
            pzs = jnp.full((L,), jnp.sum(jnp.where(sel, pzc, 0.0)))

            def dsq_at(c):
                gxv = gx[pl.ds(c * L, L)]
                gyv = gy[pl.ds(c * L, L)]
                gzv = gz[pl.ds(c * L, L)]
                dx = pxs - gxv
                dy = pys - gyv
                dz = pzs - gzv
                return dx * dx + dy * dy + dz * dz

            def mn(c, acc):
                return jnp.minimum(acc, dsq_at(c))

            acc_lt = plsc.parallel_loop(0, cb, unroll=8, carry=big)(mn)
            acc_gt = plsc.parallel_loop(cb + 1, NCHUNK, unroll=8, carry=big)(mn)
            db = dsq_at(cb)
            acc_lt = jnp.minimum(acc_lt, jnp.where(lane < rr, db, BIG))
            acc_gt = jnp.minimum(acc_gt, jnp.where(lane > rr, db, BIG))
            mlt = jnp.min(acc_lt)
            mgt = jnp.min(acc_gt)
            res_lt = jnp.where(sel, mlt, res_lt)
            res_gt = jnp.where(sel, mgt, res_gt)
            return res_lt, res_gt

        res_lt, res_gt = lax.fori_loop(0, L, row_fn, (zeros, zeros))
        lt_o[pl.ds(gbase, L)] = res_lt
        gt_o[pl.ds(gbase, L)] = res_gt
        return carry

    lax.fori_loop(0, RPW // L, group_fn, 0)
    orow = out_base // 128
    ocol = out_base % 128
    pltpu.sync_copy(lt_o, minlt_hbm.at[orow, pl.ds(ocol, RPW)])
    pltpu.sync_copy(gt_o, mingt_hbm.at[orow, pl.ds(ocol, RPW)])


@functools.cache
def _get_sc_call():
    mesh = plsc.VectorSubcoreMesh(core_axis_name="c", subcore_axis_name="s")
    return functools.partial(
        pl.kernel,
        out_type=[
            jax.ShapeDtypeStruct((N_SC // 128, 128), jnp.float32),
            jax.ShapeDtypeStruct((N_SC // 128, 128), jnp.float32),
        ],
        mesh=mesh,
        compiler_params=pltpu.CompilerParams(needs_layout_passes=False),
        scratch_types=[
            pltpu.VMEM((N,), jnp.float32),
            pltpu.VMEM((N,), jnp.float32),
            pltpu.VMEM((N,), jnp.float32),
            pltpu.VMEM((RPW,), jnp.float32),
            pltpu.VMEM((RPW,), jnp.float32),
            pltpu.VMEM((RPW,), jnp.float32),
            pltpu.VMEM((RPW,), jnp.float32),
            pltpu.VMEM((RPW,), jnp.float32),
        ],
    )(_sc_body)


def _tcdist_body(pred_ref, gt_ref, lt_ref, gt_out_ref):
    tg = pl.program_id(0)
    sub = lax.broadcasted_iota(jnp.int32, (8, 128), 0)
    lanei = lax.broadcasted_iota(jnp.int32, (8, 128), 1)
    big = jnp.full((8, 128), BIG, jnp.float32)

    for tile_sub in range(TPG):
        t = tg * TPG + tile_sub
        _tcdist_tile(pred_ref, gt_ref, lt_ref, gt_out_ref, t, sub, lanei, big)


def _tcdist_tile(pred_ref, gt_ref, lt_ref, gt_out_ref, t, sub, lanei, big):
    def pred_block(rb):
        out = []
        for d in range(3):
            rows = [pred_ref[d, t * 128 + rb * 8 + s] for s in range(8)]
            out.append(jnp.concatenate(
                [jnp.full((1, 128), v, jnp.float32) for v in rows], axis=0))
        return tuple(out)

    def gt_chunk(c):
        gxv = jnp.broadcast_to(gt_ref[0, c, :].reshape(1, 128), (8, 128))
        gyv = jnp.broadcast_to(gt_ref[1, c, :].reshape(1, 128), (8, 128))
        gzv = jnp.broadcast_to(gt_ref[2, c, :].reshape(1, 128), (8, 128))
        return gxv, gyv, gzv

    def dsq(p, g):
        dx = p[0] - g[0]
        dy = p[1] - g[1]
        dz = p[2] - g[2]
        return dx * dx + dy * dy + dz * dz

    GK = 4
    zero = jnp.float32(0.0)
    bigs = jnp.float32(BIG)
    for rbg in range(16 // GK):
        rbs = [rbg * GK + k for k in range(GK)]
        preds = [pred_block(rb) for rb in rbs]
        accs_lt = [big] * GK
        accs_gt = [big] * GK
        for c in range(N // 128):
            g = gt_chunk(c)
            pen_lt = jnp.where(c < t, zero, bigs)
            pen_gt = jnp.where(c > t, zero, bigs)
            for k in range(GK):
                d = dsq(preds[k], g)
                accs_lt[k] = jnp.minimum(accs_lt[k], d + pen_lt)
                accs_gt[k] = jnp.minimum(accs_gt[k], d + pen_gt)
        gb = gt_chunk(t)
        for k, rb in enumerate(rbs):
            db = dsq(preds[k], gb)
            acc_lt = jnp.minimum(accs_lt[k],
                                 jnp.where(lanei < rb * 8 + sub, db, BIG))
            acc_gt = jnp.minimum(accs_gt[k],
                                 jnp.where(lanei > rb * 8 + sub, db, BIG))
            lt_ref[t, pl.ds(rb * 8, 8)] = jnp.min(acc_lt, axis=1)
            gt_out_ref[t, pl.ds(rb * 8, 8)] = jnp.min(acc_gt, axis=1)


_tcdist_call = pl.pallas_call(
    _tcdist_body,
    grid=(S_TC // (128 * TPG),),
    in_specs=[
        pl.BlockSpec(memory_space=pltpu.SMEM),
        pl.BlockSpec(memory_space=pltpu.VMEM),
    ],
    out_specs=[
        pl.BlockSpec((S_TC // 128, 128), lambda t: (0, 0)),
        pl.BlockSpec((S_TC // 128, 128), lambda t: (0, 0)),
    ],
    out_shape=[
        jax.ShapeDtypeStruct((S_TC // 128, 128), jnp.float32),
        jax.ShapeDtypeStruct((S_TC // 128, 128), jnp.float32),
    ],
)


def _post_body(mlt_tc_ref, mgt_tc_ref, mlt_sc_ref, mgt_sc_ref,
               diag_ref, w_ref, lg_ref, out_ref):
    ntc = S_TC // 128
    s_ii_tc = jnp.sqrt(diag_ref[0, :ntc])
    m_tc = jnp.logical_and(jnp.sqrt(mlt_tc_ref[...]) > s_ii_tc,
                           jnp.sqrt(mgt_tc_ref[...]) >= s_ii_tc)
    s_ii_sc = jnp.sqrt(diag_ref[0, ntc:])
    m_sc = jnp.logical_and(jnp.sqrt(mlt_sc_ref[...]) > s_ii_sc,
                           jnp.sqrt(mgt_sc_ref[...]) >= s_ii_sc)
    m1 = m_tc.astype(jnp.float32)
    m2 = m_sc.astype(jnp.float32)
    x = lg_ref[...]
    bce0 = jnp.maximum(x, 0.0) + jnp.log(1.0 + jnp.exp(-jnp.abs(x)))
    bce1 = bce0 - x
    cnt1 = jnp.sum(m1) + jnp.sum(m2)
    cnt0 = jnp.float32(N) - cnt1
    s0 = (jnp.sum(bce0[:ntc] * (1.0 - m1))
          + jnp.sum(bce0[ntc:] * (1.0 - m2)))
    s1 = jnp.sum(bce1[:ntc] * m1) + jnp.sum(bce1[ntc:] * m2)
    mean0 = s0 / jnp.maximum(cnt0, 1.0)
    mean1 = s1 / jnp.maximum(cnt1, 1.0)
    inlier = (jnp.where(cnt0 > 0.0, mean0, 0.0)
              + jnp.where(cnt1 > 0.0, mean1, 0.0)) * 0.5
    w = w_ref[...]
    err = diag_ref[1]
    mean_err = jnp.sum(w * err) / jnp.maximum(jnp.sum(w), 1e-6)
    out_ref[0, 0] = mean_err + inlier


_post_call = pl.pallas_call(
    _post_body,
    out_shape=jax.ShapeDtypeStruct((1, 1), jnp.float32),
    out_specs=pl.BlockSpec(memory_space=pltpu.SMEM),
)


def kernel(kp_before, kp_warped_pred, pose_gt, overlap_weights, inlier_logits):
    kb = jnp.transpose(kp_before, (2, 0, 1)).reshape(3, N)
    pr = jnp.transpose(kp_warped_pred, (2, 0, 1)).reshape(3, N)
    pose = pose_gt.reshape(4, 12)
    coords, gt3, diag = _prep_call(pose, kb, pr)
    minlt_sc, mingt_sc = _get_sc_call()(coords)
    minlt_tc, mingt_tc = _tcdist_call(pr, gt3)
    out = _post_call(
        minlt_tc, mingt_tc, minlt_sc, mingt_sc, diag,
        overlap_weights.reshape(32, 128),
        inlier_logits.reshape(32, 128),
    )
    return out[0, 0]
